# baseline (speedup 1.0000x reference)
import functools

import jax
import jax.numpy as jnp
from jax import lax
from jax.experimental import pallas as pl
from jax.experimental.pallas import tpu as pltpu
from jax.experimental.pallas import tpu_sc as plsc

TOTAL_TOKENS = 8192
HIDDEN = 2048
INTER = 5632
B_EXPERT = 512

BLK_I = 512
N_BLK = INTER // BLK_I

_NC, _NS = 2, 16
_NW = _NC * _NS
_B_PER_W = B_EXPERT // _NW


def _gather_body(x_hbm, idx_hbm, out_hbm, idx_v, rows_v, sem):
    wid = lax.axis_index("s") * _NC + lax.axis_index("c")
    base = wid * _B_PER_W
    pltpu.sync_copy(idx_hbm.at[pl.ds(base, _B_PER_W)], idx_v)
    pltpu.async_copy(x_hbm.at[idx_v], rows_v, sem).wait()
    pltpu.sync_copy(rows_v, out_hbm.at[pl.ds(base, _B_PER_W)])


@functools.cache
def _sc_gather():
    return pl.kernel(
        _gather_body,
        out_type=jax.ShapeDtypeStruct((B_EXPERT, HIDDEN), jnp.float32),
        mesh=plsc.VectorSubcoreMesh(core_axis_name="c", subcore_axis_name="s"),
        scratch_types=[
            pltpu.VMEM((_B_PER_W,), jnp.int32),
            pltpu.VMEM((_B_PER_W, HIDDEN), jnp.float32),
            pltpu.SemaphoreType.DMA,
        ],
    )


_NT = (((1,), (1,)), ((), ()))


def _ffn_body(xs_ref, wg_ref, wu_ref, wd_ref, w_ref, out_ref, acc_ref):
    i = pl.program_id(0)
    xb = xs_ref[...]
    gab = lax.dot_general(xb, wg_ref[...].reshape(2 * BLK_I, HIDDEN), _NT,
                          preferred_element_type=jnp.float32)
    ga = gab[:, :BLK_I]
    gb = gab[:, BLK_I:]
    up = lax.dot_general(xb, wu_ref[...], _NT,
                         preferred_element_type=jnp.float32)
    h = ga * (1.0 / (1.0 + jnp.exp(-gb))) * up
    wd = wd_ref[:, pl.ds((i % 2) * BLK_I, BLK_I)]
    contrib = lax.dot_general(h, wd, _NT,
                              preferred_element_type=jnp.float32)

    @pl.when(i == 0)
    def _init():
        acc_ref[...] = contrib

    @pl.when(i > 0)
    def _accum():
        acc_ref[...] += contrib

    @pl.when(i == N_BLK - 1)
    def _scale():
        out_ref[...] = acc_ref[...] * w_ref[...]


def _tc_ffn(xs, weight, W_gate, W_up, W_down):
    return pl.pallas_call(
        _ffn_body,
        grid=(N_BLK,),
        in_specs=[
            pl.BlockSpec((B_EXPERT, HIDDEN), lambda i: (0, 0)),
            pl.BlockSpec((2, BLK_I, HIDDEN), lambda i: (0, i, 0)),
            pl.BlockSpec((BLK_I, HIDDEN), lambda i: (i, 0)),
            pl.BlockSpec((HIDDEN, 2 * BLK_I), lambda i: (0, i // 2)),
            pl.BlockSpec((B_EXPERT, 1), lambda i: (0, 0)),
        ],
        out_specs=pl.BlockSpec((B_EXPERT, HIDDEN), lambda i: (0, 0)),
        out_shape=jax.ShapeDtypeStruct((B_EXPERT, HIDDEN), jnp.float32),
        scratch_shapes=[pltpu.VMEM((B_EXPERT, HIDDEN), jnp.float32)],
        compiler_params=pltpu.CompilerParams(
            dimension_semantics=("arbitrary",),
        ),
    )(xs, W_gate.reshape(2, INTER, HIDDEN), W_up, W_down, weight)


def kernel(x, top_x, weight, W_gate, W_up, W_down):
    xs = _sc_gather()(x, top_x.astype(jnp.int32))
    return _tc_ffn(xs, weight, W_gate, W_up, W_down)


# speedup vs baseline: 1.0886x; 1.0886x over previous
import functools

import jax
import jax.numpy as jnp
from jax import lax
from jax.experimental import pallas as pl
from jax.experimental.pallas import tpu as pltpu
from jax.experimental.pallas import tpu_sc as plsc

TOTAL_TOKENS = 8192
HIDDEN = 2048
INTER = 5632
B_EXPERT = 512

BLK_I = 512
N_BLK = INTER // BLK_I

_NC, _NS = 2, 16
_NW = _NC * _NS
_B_PER_W = B_EXPERT // _NW


def _gather_body(x_hbm, idx_hbm, out_hbm, idx_v, rows_v, sem):
    wid = lax.axis_index("s") * _NC + lax.axis_index("c")
    base = wid * _B_PER_W
    pltpu.sync_copy(idx_hbm.at[pl.ds(base, _B_PER_W)], idx_v)
    pltpu.async_copy(x_hbm.at[idx_v], rows_v, sem).wait()
    pltpu.sync_copy(rows_v, out_hbm.at[pl.ds(base, _B_PER_W)])


@functools.cache
def _sc_gather():
    return pl.kernel(
        _gather_body,
        out_type=jax.ShapeDtypeStruct((B_EXPERT, HIDDEN), jnp.float32),
        mesh=plsc.VectorSubcoreMesh(core_axis_name="c", subcore_axis_name="s"),
        scratch_types=[
            pltpu.VMEM((_B_PER_W,), jnp.int32),
            pltpu.VMEM((_B_PER_W, HIDDEN), jnp.float32),
            pltpu.SemaphoreType.DMA,
        ],
    )


_NT = (((1,), (1,)), ((), ()))


def _ffn_body(xs_ref, wg_ref, wu_ref, wd_ref, w_ref, out_ref, acc_ref):
    i = pl.program_id(0)
    xb = xs_ref[...]
    gab = lax.dot_general(xb, wg_ref[...].reshape(2 * BLK_I, HIDDEN), _NT,
                          preferred_element_type=jnp.float32)
    ga = gab[:, :BLK_I]
    gb = gab[:, BLK_I:]
    up = lax.dot_general(xb, wu_ref[...], _NT,
                         preferred_element_type=jnp.float32)
    h = ga * (1.0 / (1.0 + jnp.exp(-gb))) * up
    contrib = lax.dot_general(h, wd_ref[...], _NT,
                              preferred_element_type=jnp.float32)

    @pl.when(i == 0)
    def _init():
        acc_ref[...] = contrib

    @pl.when(i > 0)
    def _accum():
        acc_ref[...] += contrib

    @pl.when(i == N_BLK - 1)
    def _scale():
        out_ref[...] = acc_ref[...] * w_ref[...]


def _tc_ffn(xs, weight, W_gate, W_up, W_down):
    return pl.pallas_call(
        _ffn_body,
        grid=(N_BLK,),
        in_specs=[
            pl.BlockSpec((B_EXPERT, HIDDEN), lambda i: (0, 0)),
            pl.BlockSpec((2, BLK_I, HIDDEN), lambda i: (0, i, 0)),
            pl.BlockSpec((BLK_I, HIDDEN), lambda i: (i, 0)),
            pl.BlockSpec((HIDDEN, BLK_I), lambda i: (0, i)),
            pl.BlockSpec((B_EXPERT, 1), lambda i: (0, 0)),
        ],
        out_specs=pl.BlockSpec((B_EXPERT, HIDDEN), lambda i: (0, 0)),
        out_shape=jax.ShapeDtypeStruct((B_EXPERT, HIDDEN), jnp.float32),
        scratch_shapes=[pltpu.VMEM((B_EXPERT, HIDDEN), jnp.float32)],
        compiler_params=pltpu.CompilerParams(
            dimension_semantics=("arbitrary",),
        ),
    )(xs, W_gate.reshape(2, INTER, HIDDEN), W_up, W_down, weight)


def kernel(x, top_x, weight, W_gate, W_up, W_down):
    xs = _sc_gather()(x, top_x.astype(jnp.int32))
    return _tc_ffn(xs, weight, W_gate, W_up, W_down)


# speedup vs baseline: 1.0944x; 1.0054x over previous
import functools

import jax
import jax.numpy as jnp
from jax import lax
from jax.experimental import pallas as pl
from jax.experimental.pallas import tpu as pltpu
from jax.experimental.pallas import tpu_sc as plsc

TOTAL_TOKENS = 8192
HIDDEN = 2048
INTER = 5632
B_EXPERT = 512

BLK_I = 512
N_BLK = INTER // BLK_I

_NC, _NS = 2, 16
_NW = _NC * _NS
_B_PER_W = B_EXPERT // _NW


def _gather_body(x_hbm, idx_hbm, out_hbm, idx_v, rows_v, sem):
    wid = lax.axis_index("s") * _NC + lax.axis_index("c")
    base = wid * _B_PER_W
    pltpu.sync_copy(idx_hbm.at[pl.ds(base, _B_PER_W)], idx_v)
    pltpu.async_copy(x_hbm.at[idx_v], rows_v, sem).wait()
    pltpu.sync_copy(rows_v, out_hbm.at[pl.ds(base, _B_PER_W)])


@functools.cache
def _sc_gather():
    return pl.kernel(
        _gather_body,
        out_type=jax.ShapeDtypeStruct((B_EXPERT, HIDDEN), jnp.float32),
        mesh=plsc.VectorSubcoreMesh(core_axis_name="c", subcore_axis_name="s"),
        scratch_types=[
            pltpu.VMEM((_B_PER_W,), jnp.int32),
            pltpu.VMEM((_B_PER_W, HIDDEN), jnp.float32),
            pltpu.SemaphoreType.DMA,
        ],
    )


_NT = (((1,), (1,)), ((), ()))


def _ffn_body(xs_ref, wg_ref, wu_ref, wd_ref, w_ref, out_ref, acc_ref):
    i = pl.program_id(0)
    xb = xs_ref[...]
    gab = lax.dot_general(xb, wg_ref[...].reshape(2 * BLK_I, HIDDEN), _NT,
                          preferred_element_type=jnp.float32)
    ga = gab[:, :BLK_I]
    gb = gab[:, BLK_I:]
    up = lax.dot_general(xb, wu_ref[...], _NT,
                         preferred_element_type=jnp.float32)
    h = ga * (1.0 / (1.0 + jnp.exp(-gb))) * up
    contrib = lax.dot_general(h, wd_ref[...], _NT,
                              preferred_element_type=jnp.float32)

    @pl.when(i == 0)
    def _init():
        acc_ref[...] = contrib

    @pl.when((i > 0) & (i < N_BLK - 1))
    def _accum():
        acc_ref[...] += contrib

    @pl.when(i == N_BLK - 1)
    def _final():
        out_ref[...] = (acc_ref[...] + contrib) * w_ref[...]


def _tc_ffn(xs, weight, W_gate, W_up, W_down):
    return pl.pallas_call(
        _ffn_body,
        grid=(N_BLK,),
        in_specs=[
            pl.BlockSpec((B_EXPERT, HIDDEN), lambda i: (0, 0)),
            pl.BlockSpec((2, BLK_I, HIDDEN), lambda i: (0, i, 0)),
            pl.BlockSpec((BLK_I, HIDDEN), lambda i: (i, 0)),
            pl.BlockSpec((HIDDEN, BLK_I), lambda i: (0, i)),
            pl.BlockSpec((B_EXPERT, 1), lambda i: (0, 0)),
        ],
        out_specs=pl.BlockSpec((B_EXPERT, HIDDEN), lambda i: (0, 0)),
        out_shape=jax.ShapeDtypeStruct((B_EXPERT, HIDDEN), jnp.float32),
        scratch_shapes=[pltpu.VMEM((B_EXPERT, HIDDEN), jnp.float32)],
        compiler_params=pltpu.CompilerParams(
            dimension_semantics=("arbitrary",),
        ),
    )(xs, W_gate.reshape(2, INTER, HIDDEN), W_up, W_down, weight)


def kernel(x, top_x, weight, W_gate, W_up, W_down):
    xs = _sc_gather()(x, top_x.astype(jnp.int32))
    return _tc_ffn(xs, weight, W_gate, W_up, W_down)
